# baseline (speedup 1.0000x reference)
import jax
import jax.numpy as jnp
from jax import lax
from jax.experimental import pallas as pl
from jax.experimental.pallas import tpu as pltpu
from jax.experimental.pallas import tpu_sc as plsc

R = 3
C = 4
NNZ = 5
COLS = 1024
NS = 16
NCHUNK = 4
L = 16
W = COLS // NCHUNK
NU = R * NCHUNK


def _bcast(v, k):
    return v.at[jnp.full((L,), k, jnp.int32)].get(mode="promise_in_bounds")


def _body(x_hbm, idx_hbm, vals_hbm, out_hbm, x_v, idx_v, vals_v, out_v,
          sem, msem):
    wid = lax.axis_index("s")

    @pl.when(wid < NU)
    def _():
        r = wid // NCHUNK
        base = (wid % NCHUNK) * W

        xcp = pltpu.async_copy(x_hbm, x_v, sem)
        mcps = [
            pltpu.async_copy(idx_hbm, idx_v.at[pl.ds(0, 2 * NNZ)], msem),
            pltpu.async_copy(vals_hbm, vals_v.at[pl.ds(0, NNZ)], msem),
        ]
        for cp in mcps:
            cp.wait()

        lane = lax.iota(jnp.int32, L)
        idx = idx_v[...]
        rows = idx
        cols = idx.at[jnp.minimum(lane + NNZ, L - 1)].get(
            mode="promise_in_bounds")
        vals = vals_v[...]
        key = rows * C + cols

        hist = jnp.zeros((L,), jnp.float32)
        for k in range(NNZ):
            hist = hist + jnp.where(_bcast(key, k) == lane,
                                    _bcast(vals, k), 0.0)

        rf = r * C
        m = [_bcast(hist, rf + c) for c in range(C)]

        xcp.wait()

        for j in range(W // L):
            xs = [x_v[pl.ds(c * COLS + base + j * L, L)] for c in range(C)]
            acc = m[0] * xs[0]
            for c in range(1, C):
                acc = acc + m[c] * xs[c]
            out_v[pl.ds(j * L, L)] = acc

        pltpu.sync_copy(out_v, out_hbm.at[pl.ds(r * COLS + base, W)])


@jax.jit
def _spmm(x_flat, idx_flat, values):
    mesh = plsc.VectorSubcoreMesh(
        core_axis_name="c", subcore_axis_name="s",
        num_cores=1, num_subcores=NS)
    out_flat = pl.kernel(
        _body,
        out_type=jax.ShapeDtypeStruct((R * COLS,), jnp.float32),
        mesh=mesh,
        scratch_types=[
            pltpu.VMEM((C * COLS,), jnp.float32),
            pltpu.VMEM((L,), jnp.int32),
            pltpu.VMEM((L,), jnp.float32),
            pltpu.VMEM((W,), jnp.float32),
            pltpu.SemaphoreType.DMA,
            pltpu.SemaphoreType.DMA,
        ],
    )(x_flat, idx_flat, values)
    return out_flat.reshape(R, COLS)


def kernel(x, indices, values):
    return _spmm(x.reshape(C * COLS), indices.reshape(2 * NNZ), values)


# speedup vs baseline: 1.0381x; 1.0381x over previous
import jax
import jax.numpy as jnp
from jax import lax
from jax.experimental import pallas as pl
from jax.experimental.pallas import tpu as pltpu
from jax.experimental.pallas import tpu_sc as plsc

R = 3
C = 4
NNZ = 5
COLS = 1024
NS = 16
NCHUNK = 4
L = 16
W = COLS // NCHUNK
NU = R * NCHUNK


def _bcast(v, k):
    return v.at[jnp.full((L,), k, jnp.int32)].get(mode="promise_in_bounds")


def _body(x_hbm, idx_hbm, vals_hbm, out_hbm, x_v, idx_v, vals_v, out_v,
          sem, msem):
    wid = lax.axis_index("s")

    @pl.when(wid < NU)
    def _():
        r = wid // NCHUNK
        base = (wid % NCHUNK) * W

        xcps = [pltpu.async_copy(
            x_hbm.at[pl.ds(c * COLS + base, W)],
            x_v.at[pl.ds(c * W, W)], sem) for c in range(C)]
        mcps = [
            pltpu.async_copy(idx_hbm, idx_v.at[pl.ds(0, 2 * NNZ)], msem),
            pltpu.async_copy(vals_hbm, vals_v.at[pl.ds(0, NNZ)], msem),
        ]
        for cp in mcps:
            cp.wait()

        lane = lax.iota(jnp.int32, L)
        idx = idx_v[...]
        rows = idx
        cols = idx.at[jnp.minimum(lane + NNZ, L - 1)].get(
            mode="promise_in_bounds")
        vals = vals_v[...]
        key = rows * C + cols

        hist = jnp.zeros((L,), jnp.float32)
        for k in range(NNZ):
            hist = hist + jnp.where(_bcast(key, k) == lane,
                                    _bcast(vals, k), 0.0)

        rf = r * C
        m = [_bcast(hist, rf + c) for c in range(C)]

        for cp in xcps:
            cp.wait()

        for j in range(W // L):
            xs = [x_v[pl.ds(c * W + j * L, L)] for c in range(C)]
            acc = m[0] * xs[0]
            for c in range(1, C):
                acc = acc + m[c] * xs[c]
            out_v[pl.ds(j * L, L)] = acc

        pltpu.sync_copy(out_v, out_hbm.at[pl.ds(r * COLS + base, W)])


@jax.jit
def _spmm(x_flat, idx_flat, values):
    mesh = plsc.VectorSubcoreMesh(
        core_axis_name="c", subcore_axis_name="s",
        num_cores=1, num_subcores=NS)
    out_flat = pl.kernel(
        _body,
        out_type=jax.ShapeDtypeStruct((R * COLS,), jnp.float32),
        mesh=mesh,
        scratch_types=[
            pltpu.VMEM((C * W,), jnp.float32),
            pltpu.VMEM((L,), jnp.int32),
            pltpu.VMEM((L,), jnp.float32),
            pltpu.VMEM((W,), jnp.float32),
            pltpu.SemaphoreType.DMA,
            pltpu.SemaphoreType.DMA,
        ],
    )(x_flat, idx_flat, values)
    return out_flat.reshape(R, COLS)


def kernel(x, indices, values):
    return _spmm(x.reshape(C * COLS), indices.reshape(2 * NNZ), values)


# speedup vs baseline: 1.0507x; 1.0121x over previous
import jax
import jax.numpy as jnp
from jax import lax
from jax.experimental import pallas as pl
from jax.experimental.pallas import tpu as pltpu
from jax.experimental.pallas import tpu_sc as plsc

R = 3
C = 4
NNZ = 5
COLS = 1024
NS = 16
L = 16
W = COLS // NS


def _bcast(v, k):
    return v.at[jnp.full((L,), k, jnp.int32)].get(mode="promise_in_bounds")


def _body(x_hbm, idx_hbm, vals_hbm, out_hbm, x_v, idx_v, vals_v, out_v,
          sem, msem, osem):
    wid = lax.axis_index("s")
    base = wid * W

    xcps = [pltpu.async_copy(
        x_hbm.at[pl.ds(c * COLS + base, W)],
        x_v.at[pl.ds(c * W, W)], sem) for c in range(C)]
    mcps = [
        pltpu.async_copy(idx_hbm, idx_v.at[pl.ds(0, 2 * NNZ)], msem),
        pltpu.async_copy(vals_hbm, vals_v.at[pl.ds(0, NNZ)], msem),
    ]
    for cp in mcps:
        cp.wait()

    lane = lax.iota(jnp.int32, L)
    idx = idx_v[...]
    rows = idx
    cols = idx.at[jnp.minimum(lane + NNZ, L - 1)].get(
        mode="promise_in_bounds")
    vals = vals_v[...]
    key = rows * C + cols

    hist = jnp.zeros((L,), jnp.float32)
    for k in range(NNZ):
        hist = hist + jnp.where(_bcast(key, k) == lane,
                                _bcast(vals, k), 0.0)

    for cp in xcps:
        cp.wait()

    for r in range(R):
        m = [_bcast(hist, r * C + c) for c in range(C)]
        for j in range(W // L):
            xs = [x_v[pl.ds(c * W + j * L, L)] for c in range(C)]
            acc = m[0] * xs[0]
            for c in range(1, C):
                acc = acc + m[c] * xs[c]
            out_v[pl.ds(r * W + j * L, L)] = acc

    ocps = [pltpu.async_copy(
        out_v.at[pl.ds(r * W, W)],
        out_hbm.at[pl.ds(r * COLS + base, W)], osem) for r in range(R)]
    for cp in ocps:
        cp.wait()


@jax.jit
def _spmm(x_flat, idx_flat, values):
    mesh = plsc.VectorSubcoreMesh(
        core_axis_name="c", subcore_axis_name="s",
        num_cores=1, num_subcores=NS)
    out_flat = pl.kernel(
        _body,
        out_type=jax.ShapeDtypeStruct((R * COLS,), jnp.float32),
        mesh=mesh,
        scratch_types=[
            pltpu.VMEM((C * W,), jnp.float32),
            pltpu.VMEM((L,), jnp.int32),
            pltpu.VMEM((L,), jnp.float32),
            pltpu.VMEM((R * W,), jnp.float32),
            pltpu.SemaphoreType.DMA,
            pltpu.SemaphoreType.DMA,
            pltpu.SemaphoreType.DMA,
        ],
    )(x_flat, idx_flat, values)
    return out_flat.reshape(R, COLS)


def kernel(x, indices, values):
    return _spmm(x.reshape(C * COLS), indices.reshape(2 * NNZ), values)


# speedup vs baseline: 1.1116x; 1.0580x over previous
import jax
import jax.numpy as jnp
from jax import lax
from jax.experimental import pallas as pl
from jax.experimental.pallas import tpu as pltpu
from jax.experimental.pallas import tpu_sc as plsc

R = 3
C = 4
NNZ = 5
COLS = 1024
NS = 16
L = 16
NW = 8
W = COLS // NW


def _bcast(v, k):
    return v.at[jnp.full((L,), k, jnp.int32)].get(mode="promise_in_bounds")


def _body(x_hbm, idx_hbm, vals_hbm, out_hbm, x_v, idx_v, vals_v, out_v,
          sem, msem, osem):
    wid = lax.axis_index("s")

    @pl.when(wid < NW)
    def _():
        base = wid * W
        xcp = pltpu.async_copy(x_hbm.at[:, pl.ds(base, W)], x_v, sem)
        mcps = [
            pltpu.async_copy(idx_hbm, idx_v.at[pl.ds(0, 2 * NNZ)], msem),
            pltpu.async_copy(vals_hbm, vals_v.at[pl.ds(0, NNZ)], msem),
        ]
        for cp in mcps:
            cp.wait()

        lane = lax.iota(jnp.int32, L)
        idx = idx_v[...]
        rows = idx
        cols = idx.at[jnp.minimum(lane + NNZ, L - 1)].get(
            mode="promise_in_bounds")
        vals = vals_v[...]
        key = rows * C + cols

        hist = jnp.zeros((L,), jnp.float32)
        for k in range(NNZ):
            hist = hist + jnp.where(_bcast(key, k) == lane,
                                    _bcast(vals, k), 0.0)

        xcp.wait()

        for r in range(R):
            m = [_bcast(hist, r * C + c) for c in range(C)]
            for j in range(W // L):
                xs = [x_v[c, pl.ds(j * L, L)] for c in range(C)]
                acc = m[0] * xs[0]
                for c in range(1, C):
                    acc = acc + m[c] * xs[c]
                out_v[r, pl.ds(j * L, L)] = acc

        ocp = pltpu.async_copy(out_v, out_hbm.at[:, pl.ds(base, W)], osem)
        ocp.wait()


@jax.jit
def _spmm(x, idx_flat, values):
    mesh = plsc.VectorSubcoreMesh(
        core_axis_name="c", subcore_axis_name="s",
        num_cores=1, num_subcores=NS)
    return pl.kernel(
        _body,
        out_type=jax.ShapeDtypeStruct((R, COLS), jnp.float32),
        mesh=mesh,
        scratch_types=[
            pltpu.VMEM((C, W), jnp.float32),
            pltpu.VMEM((L,), jnp.int32),
            pltpu.VMEM((L,), jnp.float32),
            pltpu.VMEM((R, W), jnp.float32),
            pltpu.SemaphoreType.DMA,
            pltpu.SemaphoreType.DMA,
            pltpu.SemaphoreType.DMA,
        ],
    )(x, idx_flat, values)


def kernel(x, indices, values):
    return _spmm(x, indices.reshape(2 * NNZ), values)


# speedup vs baseline: 1.1188x; 1.0065x over previous
import jax
import jax.numpy as jnp
from jax import lax
from jax.experimental import pallas as pl
from jax.experimental.pallas import tpu as pltpu
from jax.experimental.pallas import tpu_sc as plsc

R = 3
C = 4
NNZ = 5
COLS = 1024
NS = 16
L = 16
NW = 8
W = COLS // NW


def _bcast(v, k):
    return v.at[jnp.full((L,), k, jnp.int32)].get(mode="promise_in_bounds")


def _body(x_hbm, idx_hbm, vals_hbm, out_hbm, x_v, idx_v, vals_v, out_v,
          sem, msem, osem):
    wid = lax.axis_index("s")

    @pl.when(wid < NW)
    def _():
        base = wid * W
        xcp = pltpu.async_copy(x_hbm.at[:, pl.ds(base, W)], x_v, sem)
        mcps = [
            pltpu.async_copy(idx_hbm, idx_v.at[pl.ds(0, 2 * NNZ)], msem),
            pltpu.async_copy(vals_hbm, vals_v.at[pl.ds(0, NNZ)], msem),
        ]
        for cp in mcps:
            cp.wait()

        lane = lax.iota(jnp.int32, L)
        idx = idx_v[...]
        rows = idx
        cols = idx.at[jnp.minimum(lane + NNZ, L - 1)].get(
            mode="promise_in_bounds")
        vals = vals_v[...]
        key = rows * C + cols

        hist = jnp.zeros((L,), jnp.float32)
        for k in range(NNZ):
            hist = hist + jnp.where(_bcast(key, k) == lane,
                                    _bcast(vals, k), 0.0)

        xcp.wait()

        for r in range(R):
            m = [_bcast(hist, r * C + c) for c in range(C)]
            for j in range(W // L):
                xs = [x_v[c, pl.ds(j * L, L)] for c in range(C)]
                acc = m[0] * xs[0]
                for c in range(1, C):
                    acc = acc + m[c] * xs[c]
                out_v[r, pl.ds(j * L, L)] = acc

        ocp = pltpu.async_copy(out_v, out_hbm.at[:, pl.ds(base, W)], osem)
        ocp.wait()


@jax.jit
def _spmm(x, idx_flat, values):
    mesh = plsc.VectorSubcoreMesh(
        core_axis_name="c", subcore_axis_name="s",
        num_cores=1, num_subcores=NS)
    return pl.kernel(
        _body,
        out_type=jax.ShapeDtypeStruct((R, COLS), jnp.float32),
        mesh=mesh,
        compiler_params=pltpu.CompilerParams(
            skip_device_barrier=True,
            disable_semaphore_checks=True,
        ),
        scratch_types=[
            pltpu.VMEM((C, W), jnp.float32),
            pltpu.VMEM((L,), jnp.int32),
            pltpu.VMEM((L,), jnp.float32),
            pltpu.VMEM((R, W), jnp.float32),
            pltpu.SemaphoreType.DMA,
            pltpu.SemaphoreType.DMA,
            pltpu.SemaphoreType.DMA,
        ],
    )(x, idx_flat, values)


def kernel(x, indices, values):
    return _spmm(x, indices.reshape(2 * NNZ), values)
